# baseline (speedup 1.0000x reference)
import functools

import jax
import jax.numpy as jnp
from jax import lax
from jax.experimental import pallas as pl
from jax.experimental.pallas import tpu as pltpu
from jax.experimental.pallas import tpu_sc as plsc

VOCAB = 1000000
D = 64
CHUNK = 128


def _make_gather(n_rows: int):
    info = plsc.get_sparse_core_info()
    nw = info.num_cores * info.num_subcores
    n_chunks = n_rows // CHUNK
    cpw = n_chunks // nw

    mesh = plsc.VectorSubcoreMesh(core_axis_name="c", subcore_axis_name="s")

    @functools.partial(
        pl.kernel,
        mesh=mesh,
        out_type=jax.ShapeDtypeStruct((n_rows, D), jnp.float32),
        scratch_types=[
            pltpu.VMEM((cpw, CHUNK), jnp.int32),
            pltpu.VMEM((CHUNK, D), jnp.float32),
            pltpu.SemaphoreType.DMA,
        ],
        compiler_params=pltpu.CompilerParams(use_tc_tiling_on_sc=False),
    )
    def gather(idx_hbm, table_hbm, out_hbm, idx_v, rows_v, sem):
        wid = lax.axis_index("s") * info.num_cores + lax.axis_index("c")
        base = wid * cpw
        pltpu.sync_copy(idx_hbm.at[wid], idx_v)

        def step(j, carry):
            pltpu.async_copy(table_hbm.at[idx_v.at[j]], rows_v, sem).wait()
            start = pl.multiple_of((base + j) * CHUNK, CHUNK)
            pltpu.sync_copy(rows_v, out_hbm.at[pl.ds(start, CHUNK)])
            return carry

        lax.fori_loop(0, cpw, step, 0)

    return gather


def kernel(token_ids, weight):
    b, s = token_ids.shape
    n_rows = b * s
    nw = 32
    cpw = n_rows // (nw * CHUNK)
    idx = token_ids.reshape(nw, cpw, CHUNK).astype(jnp.int32)
    out = _make_gather(n_rows)(idx, weight)
    return out.reshape(b, s, D)


# speedup vs baseline: 1.0418x; 1.0418x over previous
import functools

import jax
import jax.numpy as jnp
from jax import lax
from jax.experimental import pallas as pl
from jax.experimental.pallas import tpu as pltpu
from jax.experimental.pallas import tpu_sc as plsc

D = 64
CHUNK = 128
K = 5
NSLOT = 2


def _make_gather(n_rows: int):
    info = plsc.get_sparse_core_info()
    nw = info.num_cores * info.num_subcores
    cpw = n_rows // (nw * CHUNK)
    ng = cpw // K
    rows_per_group = K * CHUNK

    mesh = plsc.VectorSubcoreMesh(core_axis_name="c", subcore_axis_name="s")

    @functools.partial(
        pl.kernel,
        mesh=mesh,
        out_type=jax.ShapeDtypeStruct((n_rows, D), jnp.float32),
        scratch_types=[
            pltpu.VMEM((cpw, CHUNK), jnp.int32),
            pltpu.VMEM((NSLOT, rows_per_group, D), jnp.float32),
            [pltpu.SemaphoreType.DMA] * NSLOT,
            [pltpu.SemaphoreType.DMA] * NSLOT,
        ],
        compiler_params=pltpu.CompilerParams(use_tc_tiling_on_sc=False),
    )
    def gather(idx_hbm, table_hbm, out_hbm, idx_v, rows_v, semg, semw):
        wid = lax.axis_index("s") * info.num_cores + lax.axis_index("c")
        base = wid * cpw
        pltpu.sync_copy(idx_hbm.at[wid], idx_v)

        def fire_gathers(g, slot):
            for k in range(K):
                pltpu.async_copy(
                    table_hbm.at[idx_v.at[g * K + k]],
                    rows_v.at[slot, pl.ds(k * CHUNK, CHUNK)],
                    semg[slot],
                )

        def wait_gathers(slot):
            pltpu.make_async_copy(
                out_hbm.at[pl.ds(0, rows_per_group)],
                rows_v.at[slot],
                semg[slot],
            ).wait()

        def out_slice(g):
            start = pl.multiple_of((base + g * K) * CHUNK, CHUNK)
            return out_hbm.at[pl.ds(start, rows_per_group)]

        def start_writeback(g, slot):
            pltpu.async_copy(rows_v.at[slot], out_slice(g), semw[slot])

        def wait_writeback(g, slot):
            pltpu.make_async_copy(rows_v.at[slot], out_slice(g), semw[slot]).wait()

        fire_gathers(0, 0)
        fire_gathers(1, 1)

        def step(i, carry):
            g0, g1 = 2 * i, 2 * i + 1
            wait_gathers(0)
            start_writeback(g0, 0)
            wait_gathers(1)
            start_writeback(g1, 1)
            wait_writeback(g0, 0)
            fire_gathers(g0 + 2, 0)
            wait_writeback(g1, 1)
            fire_gathers(g1 + 2, 1)
            return carry

        lax.fori_loop(0, ng // 2 - 1, step, 0)

        g0, g1 = ng - 2, ng - 1
        wait_gathers(0)
        start_writeback(g0, 0)
        wait_gathers(1)
        start_writeback(g1, 1)
        wait_writeback(g0, 0)
        wait_writeback(g1, 1)

    return gather


def kernel(token_ids, weight):
    b, s = token_ids.shape
    n_rows = b * s
    nw = 32
    cpw = n_rows // (nw * CHUNK)
    idx = token_ids.reshape(nw, cpw, CHUNK).astype(jnp.int32)
    out = _make_gather(n_rows)(idx, weight)
    return out.reshape(b, s, D)
